# baseline (speedup 1.0000x reference)
import functools

import jax
import jax.numpy as jnp
import numpy as np
from jax import lax
from jax.experimental import pallas as pl
from jax.experimental.pallas import tpu as pltpu
from jax.experimental.pallas import tpu_sc as plsc

D_MODEL = 768
CONTEXT = 4096
LANES = 16


def _make_posits_np():
    position = np.arange(0, CONTEXT, dtype=np.float32)[:, None]
    v_emb = np.arange(0, D_MODEL, 2, dtype=np.float32)
    angles = (position / (10000.0 ** (v_emb / np.float32(D_MODEL)))).astype(np.float32)
    posits = np.zeros((CONTEXT, D_MODEL), dtype=np.float32)
    posits[:, 0::2] = np.sin(angles)
    posits[:, 1::2] = np.cos(angles)
    return posits


_POSITS = _make_posits_np()


def kernel(x, table):
    B, S = x.shape
    V, D = table.shape
    N = B * S
    d_vecs = D // LANES

    info = plsc.get_sparse_core_info()
    NW = info.num_cores * info.num_subcores
    s_per_w = S // NW
    Cs = 8
    n_j = s_per_w // Cs
    NSET = 4
    PF = 2

    posits = jnp.asarray(_POSITS[:S].reshape(-1))

    mesh = plsc.VectorSubcoreMesh(core_axis_name="c", subcore_axis_name="s")

    @functools.partial(
        pl.kernel,
        mesh=mesh,
        out_type=jax.ShapeDtypeStruct((N, D), jnp.float32),
        scratch_types=[
            pltpu.VMEM((B, s_per_w), jnp.int32),
            pltpu.VMEM((NSET * B, Cs, D), jnp.float32),
            pltpu.VMEM((NSET, Cs * D), jnp.float32),
            pltpu.SemaphoreType.DMA,
            pltpu.SemaphoreType.DMA,
            pltpu.SemaphoreType.DMA,
        ],
    )
    def emb_kernel(x_hbm, tab_hbm, pos_hbm, out_hbm,
                   idx_v, rows_v, pos_v, gsem, ssem, psem):
        wid = lax.axis_index("s") * info.num_cores + lax.axis_index("c")
        s_base = wid * s_per_w

        def issue_chunk(j, st):
            pltpu.async_copy(
                pos_hbm.at[pl.ds((s_base + j * Cs) * D, Cs * D)],
                pos_v.at[st], psem)
            for b in range(B):
                pltpu.async_copy(
                    tab_hbm.at[idx_v.at[b, pl.ds(j * Cs, Cs)]],
                    rows_v.at[st * B + b], gsem)

        def wait_gathers():
            for _ in range(B):
                pltpu.make_async_copy(
                    tab_hbm.at[idx_v.at[0, pl.ds(0, Cs)]], rows_v.at[0], gsem
                ).wait()

        def wait_one_store():
            pltpu.make_async_copy(
                rows_v.at[0], out_hbm.at[pl.ds(0, Cs)], ssem
            ).wait()

        def wait_one_pos():
            pltpu.make_async_copy(
                pos_hbm.at[pl.ds(0, Cs * D)], pos_v.at[0], psem
            ).wait()

        for b in range(B):
            pltpu.sync_copy(x_hbm.at[b, pl.ds(s_base, s_per_w)], idx_v.at[b])

        for j in range(PF):
            issue_chunk(j, j % NSET)

        @pl.loop(0, n_j, step=NSET)
        def jj_body(jj):
            for dj in range(NSET):
                j = jj + dj
                st = dj

                @pl.when(j + PF < n_j)
                def _():
                    @pl.when(j + PF >= NSET)
                    def _():
                        for _ in range(B):
                            wait_one_store()

                    issue_chunk(j + PF, (st + PF) % NSET)

                wait_gathers()
                wait_one_pos()

                @pl.loop(0, Cs)
                def row_body(r, _st=st):
                    for d in range(d_vecs):
                        sl = pl.ds(d * LANES, LANES)
                        pv = pos_v[_st, pl.ds(r * D + d * LANES, LANES)]
                        for b in range(B):
                            k = _st * B + b
                            rows_v[k, r, sl] = rows_v[k, r, sl] + pv

                for b in range(B):
                    off = b * S + s_base + j * Cs
                    pltpu.async_copy(
                        rows_v.at[st * B + b],
                        out_hbm.at[pl.ds(off, Cs)], ssem)

        n_inloop = B * max(0, (n_j - PF) - max(0, NSET - PF))
        for _ in range(B * n_j - n_inloop):
            wait_one_store()

    out = emb_kernel(x, table, posits)
    return out.reshape(B, S, D)


# speedup vs baseline: 1.0977x; 1.0977x over previous
import functools

import jax
import jax.numpy as jnp
import numpy as np
from jax import lax
from jax.experimental import pallas as pl
from jax.experimental.pallas import tpu as pltpu
from jax.experimental.pallas import tpu_sc as plsc

D_MODEL = 768
CONTEXT = 4096
LANES = 16


def _make_posits_np():
    position = np.arange(0, CONTEXT, dtype=np.float32)[:, None]
    v_emb = np.arange(0, D_MODEL, 2, dtype=np.float32)
    angles = (position / (10000.0 ** (v_emb / np.float32(D_MODEL)))).astype(np.float32)
    posits = np.zeros((CONTEXT, D_MODEL), dtype=np.float32)
    posits[:, 0::2] = np.sin(angles)
    posits[:, 1::2] = np.cos(angles)
    return posits


_POSITS = _make_posits_np()


def kernel(x, table):
    B, S = x.shape
    V, D = table.shape
    N = B * S
    d_vecs = D // LANES

    info = plsc.get_sparse_core_info()
    NW = info.num_cores * info.num_subcores
    s_per_w = S // NW
    Cs = 8
    n_j = s_per_w // Cs
    CR = B * Cs
    NSET = 4
    PF = 2

    posits = jnp.asarray(_POSITS[:S])
    xt = x.reshape(B, NW, n_j, Cs).transpose(1, 2, 0, 3).reshape(NW, n_j * CR)

    mesh = plsc.VectorSubcoreMesh(core_axis_name="c", subcore_axis_name="s")

    @functools.partial(
        pl.kernel,
        mesh=mesh,
        out_type=jax.ShapeDtypeStruct((N, D), jnp.float32),
        scratch_types=[
            pltpu.VMEM((n_j * CR,), jnp.int32),
            pltpu.VMEM((NSET, CR, D), jnp.float32),
            pltpu.VMEM((NSET, Cs, D), jnp.float32),
            pltpu.SemaphoreType.DMA,
            pltpu.SemaphoreType.DMA,
            pltpu.SemaphoreType.DMA,
        ],
    )
    def emb_kernel(xt_hbm, tab_hbm, pos_hbm, out_hbm,
                   idx_v, rows_v, pos_v, gsem, ssem, psem):
        wid = lax.axis_index("s") * info.num_cores + lax.axis_index("c")
        s_base = wid * s_per_w

        def issue_chunk(j, st):
            pltpu.async_copy(
                pos_hbm.at[pl.ds(s_base + j * Cs, Cs)], pos_v.at[st], psem)
            pltpu.async_copy(
                tab_hbm.at[idx_v.at[pl.ds(j * CR, CR)]], rows_v.at[st], gsem)

        def wait_gather():
            pltpu.make_async_copy(
                tab_hbm.at[idx_v.at[pl.ds(0, CR)]], rows_v.at[0], gsem
            ).wait()

        def wait_chunk_stores():
            pltpu.make_async_copy(
                rows_v.at[0], out_hbm.at[pl.ds(0, CR)], ssem
            ).wait()

        def wait_one_pos():
            pltpu.make_async_copy(
                pos_hbm.at[pl.ds(0, Cs)], pos_v.at[0], psem
            ).wait()

        pltpu.sync_copy(xt_hbm.at[wid], idx_v)

        for j in range(PF):
            issue_chunk(j, j % NSET)

        @pl.loop(0, n_j, step=NSET)
        def jj_body(jj):
            for dj in range(NSET):
                j = jj + dj
                st = dj

                @pl.when(j + PF < n_j)
                def _():
                    @pl.when(j + PF >= NSET)
                    def _():
                        wait_chunk_stores()

                    issue_chunk(j + PF, (st + PF) % NSET)

                wait_gather()
                wait_one_pos()

                @pl.loop(0, Cs)
                def row_body(r, _st=st):
                    for d in range(d_vecs):
                        sl = pl.ds(d * LANES, LANES)
                        pv = pos_v[_st, r, sl]
                        for b in range(B):
                            rows_v[_st, b * Cs + r, sl] = (
                                rows_v[_st, b * Cs + r, sl] + pv)

                for b in range(B):
                    off = b * S + s_base + j * Cs
                    pltpu.async_copy(
                        rows_v.at[st, pl.ds(b * Cs, Cs)],
                        out_hbm.at[pl.ds(off, Cs)], ssem)

        n_inloop = max(0, (n_j - PF) - max(0, NSET - PF))
        for _ in range(n_j - n_inloop):
            wait_chunk_stores()

    out = emb_kernel(xt, table, posits)
    return out.reshape(B, S, D)


# speedup vs baseline: 1.1020x; 1.0039x over previous
import functools

import jax
import jax.numpy as jnp
import numpy as np
from jax import lax
from jax.experimental import pallas as pl
from jax.experimental.pallas import tpu as pltpu
from jax.experimental.pallas import tpu_sc as plsc

D_MODEL = 768
CONTEXT = 4096
LANES = 16


def _make_posits_np():
    position = np.arange(0, CONTEXT, dtype=np.float32)[:, None]
    v_emb = np.arange(0, D_MODEL, 2, dtype=np.float32)
    angles = (position / (10000.0 ** (v_emb / np.float32(D_MODEL)))).astype(np.float32)
    posits = np.zeros((CONTEXT, D_MODEL), dtype=np.float32)
    posits[:, 0::2] = np.sin(angles)
    posits[:, 1::2] = np.cos(angles)
    return posits


_POSITS = jax.device_put(_make_posits_np())


def kernel(x, table):
    B, S = x.shape
    V, D = table.shape
    N = B * S
    d_vecs = D // LANES

    info = plsc.get_sparse_core_info()
    NW = info.num_cores * info.num_subcores
    s_per_w = S // NW
    Cs = 8
    n_j = s_per_w // Cs
    CR = B * Cs
    NSET = 4
    PF = 2

    posits = _POSITS if S == CONTEXT else _POSITS[:S]
    xt = x.reshape(B, NW, n_j, Cs).transpose(1, 2, 0, 3).reshape(NW, n_j * CR)

    mesh = plsc.VectorSubcoreMesh(core_axis_name="c", subcore_axis_name="s")

    @functools.partial(
        pl.kernel,
        mesh=mesh,
        out_type=jax.ShapeDtypeStruct((N, D), jnp.float32),
        scratch_types=[
            pltpu.VMEM((n_j * CR,), jnp.int32),
            pltpu.VMEM((NSET, CR, D), jnp.float32),
            pltpu.VMEM((NSET, Cs, D), jnp.float32),
            pltpu.SemaphoreType.DMA,
            pltpu.SemaphoreType.DMA,
            pltpu.SemaphoreType.DMA,
        ],
    )
    def emb_kernel(xt_hbm, tab_hbm, pos_hbm, out_hbm,
                   idx_v, rows_v, pos_v, gsem, ssem, psem):
        wid = lax.axis_index("s") * info.num_cores + lax.axis_index("c")
        s_base = wid * s_per_w

        def issue_chunk(j, st):
            pltpu.async_copy(
                pos_hbm.at[pl.ds(s_base + j * Cs, Cs)], pos_v.at[st], psem)
            pltpu.async_copy(
                tab_hbm.at[idx_v.at[pl.ds(j * CR, CR)]], rows_v.at[st], gsem)

        def wait_gather():
            pltpu.make_async_copy(
                tab_hbm.at[idx_v.at[pl.ds(0, CR)]], rows_v.at[0], gsem
            ).wait()

        def wait_chunk_stores():
            pltpu.make_async_copy(
                rows_v.at[0], out_hbm.at[pl.ds(0, CR)], ssem
            ).wait()

        def wait_one_pos():
            pltpu.make_async_copy(
                pos_hbm.at[pl.ds(0, Cs)], pos_v.at[0], psem
            ).wait()

        pltpu.sync_copy(xt_hbm.at[wid], idx_v)

        for j in range(PF):
            issue_chunk(j, j % NSET)

        @pl.loop(0, n_j, step=NSET)
        def jj_body(jj):
            for dj in range(NSET):
                j = jj + dj
                st = dj

                @pl.when(j + PF < n_j)
                def _():
                    @pl.when(j + PF >= NSET)
                    def _():
                        wait_chunk_stores()

                    issue_chunk(j + PF, (st + PF) % NSET)

                wait_gather()
                wait_one_pos()

                @pl.loop(0, Cs)
                def row_body(r, _st=st):
                    for d in range(d_vecs):
                        sl = pl.ds(d * LANES, LANES)
                        pv = pos_v[_st, r, sl]
                        for b in range(B):
                            rows_v[_st, b * Cs + r, sl] = (
                                rows_v[_st, b * Cs + r, sl] + pv)

                for b in range(B):
                    off = b * S + s_base + j * Cs
                    pltpu.async_copy(
                        rows_v.at[st, pl.ds(b * Cs, Cs)],
                        out_hbm.at[pl.ds(off, Cs)], ssem)

        n_inloop = max(0, (n_j - PF) - max(0, NSET - PF))
        for _ in range(n_j - n_inloop):
            wait_chunk_stores()

    out = emb_kernel(xt, table, posits)
    return out.reshape(B, S, D)


# speedup vs baseline: 1.1179x; 1.0144x over previous
import functools

import jax
import jax.numpy as jnp
import numpy as np
from jax import lax
from jax.experimental import pallas as pl
from jax.experimental.pallas import tpu as pltpu
from jax.experimental.pallas import tpu_sc as plsc

D_MODEL = 768
CONTEXT = 4096
LANES = 16


def _make_posits_np():
    position = np.arange(0, CONTEXT, dtype=np.float32)[:, None]
    v_emb = np.arange(0, D_MODEL, 2, dtype=np.float32)
    angles = (position / (10000.0 ** (v_emb / np.float32(D_MODEL)))).astype(np.float32)
    posits = np.zeros((CONTEXT, D_MODEL), dtype=np.float32)
    posits[:, 0::2] = np.sin(angles)
    posits[:, 1::2] = np.cos(angles)
    return posits


_POSITS = _make_posits_np()


def kernel(x, table):
    B, S = x.shape
    V, D = table.shape
    N = B * S
    d_vecs = D // LANES

    info = plsc.get_sparse_core_info()
    NW = info.num_cores * info.num_subcores
    s_per_w = S // NW
    Cs = 8
    n_j = s_per_w // Cs
    CR = B * Cs
    NSET = 4
    PF = 2

    posits = jnp.asarray(_POSITS[:S])

    mesh = plsc.VectorSubcoreMesh(core_axis_name="c", subcore_axis_name="s")

    @functools.partial(
        pl.kernel,
        mesh=mesh,
        out_type=jax.ShapeDtypeStruct((N, D), jnp.float32),
        scratch_types=[
            pltpu.VMEM((B, s_per_w), jnp.int32),
            pltpu.VMEM((NSET, CR, D), jnp.float32),
            pltpu.VMEM((NSET, Cs, D), jnp.float32),
            pltpu.SemaphoreType.DMA,
            pltpu.SemaphoreType.DMA,
            pltpu.SemaphoreType.DMA,
        ],
    )
    def emb_kernel(x_hbm, tab_hbm, pos_hbm, out_hbm,
                   idx_v, rows_v, pos_v, gsem, ssem, psem):
        wid = lax.axis_index("s") * info.num_cores + lax.axis_index("c")
        s_base = wid * s_per_w

        def issue_chunk(j, st):
            pltpu.async_copy(
                pos_hbm.at[pl.ds(s_base + j * Cs, Cs)], pos_v.at[st], psem)
            for b in range(B):
                pltpu.async_copy(
                    tab_hbm.at[idx_v.at[b, pl.ds(j * Cs, Cs)]],
                    rows_v.at[st, pl.ds(b * Cs, Cs)], gsem)

        def wait_chunk_gathers():
            pltpu.make_async_copy(
                tab_hbm.at[idx_v.at[0, pl.ds(0, CR)]], rows_v.at[0], gsem
            ).wait()

        def wait_chunk_stores():
            pltpu.make_async_copy(
                rows_v.at[0], out_hbm.at[pl.ds(0, CR)], ssem
            ).wait()

        def wait_one_pos():
            pltpu.make_async_copy(
                pos_hbm.at[pl.ds(0, Cs)], pos_v.at[0], psem
            ).wait()

        for b in range(B):
            pltpu.sync_copy(x_hbm.at[b, pl.ds(s_base, s_per_w)], idx_v.at[b])

        for j in range(PF):
            issue_chunk(j, j % NSET)

        @pl.loop(0, n_j, step=NSET)
        def jj_body(jj):
            for dj in range(NSET):
                j = jj + dj
                st = dj

                @pl.when(j + PF < n_j)
                def _():
                    @pl.when(j + PF >= NSET)
                    def _():
                        wait_chunk_stores()

                    issue_chunk(j + PF, (st + PF) % NSET)

                wait_chunk_gathers()
                wait_one_pos()

                @pl.loop(0, Cs)
                def row_body(r, _st=st):
                    for d in range(d_vecs):
                        sl = pl.ds(d * LANES, LANES)
                        pv = pos_v[_st, r, sl]
                        for b in range(B):
                            rows_v[_st, b * Cs + r, sl] = (
                                rows_v[_st, b * Cs + r, sl] + pv)

                for b in range(B):
                    off = b * S + s_base + j * Cs
                    pltpu.async_copy(
                        rows_v.at[st, pl.ds(b * Cs, Cs)],
                        out_hbm.at[pl.ds(off, Cs)], ssem)

        n_inloop = max(0, (n_j - PF) - max(0, NSET - PF))
        for _ in range(n_j - n_inloop):
            wait_chunk_stores()

    out = emb_kernel(x, table, posits)
    return out.reshape(B, S, D)
